# baseline (speedup 1.0000x reference)
import functools

import jax
import jax.numpy as jnp
from jax import lax
from jax.experimental import pallas as pl
from jax.experimental.pallas import tpu as pltpu
from jax.experimental.pallas import tpu_sc as plsc

_NW = 32
_BPC = 2
_NBUF = 4


@functools.partial(jax.jit, static_argnums=(2, 3, 4))
def _emb_lookup(idx3, table, nb, hist, d):
    per_w = nb // (_NW * _BPC)
    rows = _BPC * hist
    mesh = plsc.VectorSubcoreMesh(core_axis_name="c", subcore_axis_name="s")

    @functools.partial(
        pl.kernel,
        out_type=jax.ShapeDtypeStruct((nb, hist, d), jnp.float32),
        mesh=mesh,
        scratch_types=[
            pltpu.VMEM((per_w, rows), jnp.int32),
            pltpu.VMEM((_NBUF, rows, d), jnp.float32),
            [pltpu.SemaphoreType.DMA] * _NBUF,
            [pltpu.SemaphoreType.DMA] * _NBUF,
        ],
        compiler_params=pltpu.CompilerParams(use_tc_tiling_on_sc=False),
    )
    def emb(table_hbm, idx_hbm, out_hbm, idx_v, rows_v, gsems, wsems):
        wid = lax.axis_index("s") * 2 + lax.axis_index("c")
        base_b = wid * (per_w * _BPC)
        pltpu.sync_copy(idx_hbm.at[wid], idx_v)

        def start_gather(j, b):
            pltpu.async_copy(table_hbm.at[idx_v.at[j]], rows_v.at[b], gsems[b])

        def wait_gather(j, b):
            pltpu.make_async_copy(
                table_hbm.at[idx_v.at[j]], rows_v.at[b], gsems[b]
            ).wait()

        def writes(j, b):
            out = []
            for k in range(_BPC):
                out.append(
                    pltpu.make_async_copy(
                        rows_v.at[b, pl.ds(k * hist, hist)],
                        out_hbm.at[base_b + j * _BPC + k],
                        wsems[b],
                    )
                )
            return out

        for b in range(_NBUF):
            start_gather(b, b)

        n_groups = per_w // _NBUF

        def group(g, carry):
            j0 = g * _NBUF
            for b in range(_NBUF):
                wait_gather(j0 + b, b)
                for w in writes(j0 + b, b):
                    w.start()
            for b in range(_NBUF):
                for w in writes(j0 + b, b):
                    w.wait()

                @pl.when(g < n_groups - 1)
                def _():
                    start_gather(j0 + b + _NBUF, b)

            return carry

        lax.fori_loop(0, n_groups, group, None)

    return emb(table, idx3)


def kernel(indices, table):
    nb, hist = indices.shape
    _, d = table.shape
    per_w = nb // (_NW * _BPC)
    assert per_w * _NW * _BPC == nb
    idx3 = indices.reshape(_NW, per_w, _BPC * hist)
    return _emb_lookup(idx3, table, nb, hist, d)
